# baseline (speedup 1.0000x reference)
import functools

import numpy as np

import jax
import jax.numpy as jnp
from jax import lax
from jax.experimental import pallas as pl
from jax.experimental.pallas import tpu as pltpu
import jax.experimental.pallas.tpu_sc as plsc

N = 10000
F = 128
HID = 32
NH = 4
NCLS = 32
ALPHA = 0.2
E = 320000

UW = 144
VW = 48

B = 128
SROWS = 10112

TB = 80
EPAD = 32 * TB * B

RB = 400
GRID = N // RB

ZROWS = SROWS // 16
OLAST = N - 15 * ZROWS


def _zero_fill(buf, acc, s):
    width = buf.shape[1]

    def zrow(r, carry):
        def zcol(j, carry2):
            buf[r, pl.ds(j * 16, 16)] = jnp.zeros((16,), jnp.float32)
            return carry2
        return lax.fori_loop(0, width // 16, zcol, carry)
    lax.fori_loop(0, B, zrow, 0)

    zbase = s * ZROWS
    for k in range(ZROWS // B):
        pltpu.sync_copy(buf, acc.at[pl.ds(zbase + k * B, B)])
    rem = ZROWS % B
    if rem:
        pltpu.sync_copy(buf.at[pl.ds(0, rem)],
                        acc.at[pl.ds(zbase + (ZROWS // B) * B, rem)])


def _edge_loop(table, acc, src_v, dst_v, bufs, sems, steps):
    nb = len(bufs)
    if nb == 1:
        def body1(t, carry):
            pltpu.async_copy(table.at[dst_v.at[t]], bufs[0], sems[0]).wait()
            pltpu.sync_copy(bufs[0], acc.at[src_v.at[t]], add=True)
            return carry
        lax.fori_loop(0, steps, body1, 0)
        return

    for b in range(nb):
        pltpu.async_copy(table.at[dst_v.at[b]], bufs[b], sems[b])

    def body(g, carry):
        for b in range(nb):
            t = g * nb + b
            pltpu.make_async_copy(table.at[dst_v.at[t]], bufs[b],
                                  sems[b]).wait()
            pltpu.sync_copy(bufs[b], acc.at[src_v.at[t]], add=True)

            @pl.when(t + nb < steps)
            def _():
                pltpu.async_copy(table.at[dst_v.at[t + nb]], bufs[b], sems[b])
        return carry
    lax.fori_loop(0, steps // nb, body, 0)


def _copy_out(acc, out, s):
    @pl.when(s < 15)
    def _():
        ob = s * ZROWS
        pltpu.sync_copy(acc.at[pl.ds(ob, ZROWS)], out.at[pl.ds(ob, ZROWS)])

    @pl.when(s == 15)
    def _():
        pltpu.sync_copy(acc.at[pl.ds(15 * ZROWS, OLAST)],
                        out.at[pl.ds(15 * ZROWS, OLAST)])


def _sc_pass_a():
    mesh = plsc.VectorSubcoreMesh(core_axis_name="c", subcore_axis_name="s")
    out_sd = jax.ShapeDtypeStruct((N, UW), jnp.float32)

    @functools.partial(
        pl.kernel,
        out_type=(out_sd, out_sd),
        mesh=mesh,
        scratch_types=[
            pltpu.VMEM((TB, B), jnp.int32),
            pltpu.VMEM((TB, B), jnp.int32),
            pltpu.VMEM((B, UW), jnp.float32),
            pltpu.VMEM_SHARED((SROWS, UW), jnp.float32),
            pltpu.SemaphoreType.DMA,
        ],
        compiler_params=pltpu.CompilerParams(use_tc_tiling_on_sc=False),
    )
    def sc_a(table, src_hbm, dst_hbm, out0, out1, src_v, dst_v, buf, acc, sem):
        c = lax.axis_index("c")
        s = lax.axis_index("s")
        wid = c * 16 + s

        _zero_fill(buf, acc, s)
        plsc.subcore_barrier()

        pltpu.sync_copy(src_hbm.at[wid], src_v)
        pltpu.sync_copy(dst_hbm.at[wid], dst_v)

        _edge_loop(table, acc, src_v, dst_v, (buf,), (sem,), TB)

        plsc.subcore_barrier()

        @pl.when(c == 0)
        def _():
            _copy_out(acc, out0, s)

        @pl.when(c == 1)
        def _():
            _copy_out(acc, out1, s)

    return sc_a


def _sc_pass_b():
    mesh = plsc.VectorSubcoreMesh(core_axis_name="c", subcore_axis_name="s")
    out_sd = jax.ShapeDtypeStruct((N, VW), jnp.float32)

    @functools.partial(
        pl.kernel,
        out_type=(out_sd, out_sd),
        mesh=mesh,
        scratch_types=[
            pltpu.VMEM((TB, B), jnp.int32),
            pltpu.VMEM((TB, B), jnp.int32),
            pltpu.VMEM((B, VW), jnp.float32),
            pltpu.VMEM((B, VW), jnp.float32),
            pltpu.VMEM((B, VW), jnp.float32),
            pltpu.VMEM((B, VW), jnp.float32),
            pltpu.VMEM_SHARED((SROWS, VW), jnp.float32),
            pltpu.SemaphoreType.DMA,
            pltpu.SemaphoreType.DMA,
            pltpu.SemaphoreType.DMA,
            pltpu.SemaphoreType.DMA,
        ],
        compiler_params=pltpu.CompilerParams(use_tc_tiling_on_sc=False),
    )
    def sc_b(table, src_hbm, dst_hbm, out0, out1, src_v, dst_v, b0, b1, b2, b3,
             acc, s0, s1, s2, s3):
        c = lax.axis_index("c")
        s = lax.axis_index("s")
        wid = c * 16 + s

        _zero_fill(b0, acc, s)
        plsc.subcore_barrier()

        pltpu.sync_copy(src_hbm.at[wid], src_v)
        pltpu.sync_copy(dst_hbm.at[wid], dst_v)

        _edge_loop(table, acc, src_v, dst_v, (b0, b1, b2, b3),
                   (s0, s1, s2, s3), TB)

        plsc.subcore_barrier()

        @pl.when(c == 0)
        def _():
            _copy_out(acc, out0, s)

        @pl.when(c == 1)
        def _():
            _copy_out(acc, out1, s)

    return sc_b


_sc_scatter_a = _sc_pass_a()
_sc_scatter_b = _sc_pass_b()


def _tc_dense1(x, wcat, a2):
    def body(x_ref, w_ref, a_ref, wh_ref, e2_ref):
        z = jnp.dot(x_ref[...], w_ref[...], preferred_element_type=jnp.float32)
        wh = jnp.where(z > 0, z, ALPHA * z)
        wh_ref[...] = wh
        e2_ref[...] = jnp.dot(wh, a_ref[...], preferred_element_type=jnp.float32)

    return pl.pallas_call(
        body,
        grid=(GRID,),
        in_specs=[
            pl.BlockSpec((RB, F), lambda i: (i, 0)),
            pl.BlockSpec((F, F), lambda i: (0, 0)),
            pl.BlockSpec((F, 8), lambda i: (0, 0)),
        ],
        out_specs=[
            pl.BlockSpec((RB, F), lambda i: (i, 0)),
            pl.BlockSpec((RB, 8), lambda i: (i, 0)),
        ],
        out_shape=[
            jax.ShapeDtypeStruct((N, F), jnp.float32),
            jax.ShapeDtypeStruct((N, 8), jnp.float32),
        ],
    )(x, wcat, a2)


def _tc_build_u(wh, e2, psel, esc, eden):
    def body(wh_ref, e2b_ref, e2f_ref, p_ref, esc_ref, eden_ref, u_ref):
        cmax = jnp.max(e2f_ref[...], axis=0)
        g = jnp.exp(e2b_ref[...] - cmax[None, :])
        feat = jnp.dot(wh_ref[...], p_ref[...],
                       preferred_element_type=jnp.float32)
        scale = jnp.dot(g, esc_ref[...], preferred_element_type=jnp.float32)
        den = jnp.dot(g, eden_ref[...], preferred_element_type=jnp.float32)
        u_ref[...] = feat * scale + den

    return pl.pallas_call(
        body,
        grid=(GRID,),
        in_specs=[
            pl.BlockSpec((RB, F), lambda i: (i, 0)),
            pl.BlockSpec((RB, 8), lambda i: (i, 0)),
            pl.BlockSpec((N, 8), lambda i: (0, 0)),
            pl.BlockSpec((F, UW), lambda i: (0, 0)),
            pl.BlockSpec((8, UW), lambda i: (0, 0)),
            pl.BlockSpec((8, UW), lambda i: (0, 0)),
        ],
        out_specs=pl.BlockSpec((RB, UW), lambda i: (i, 0)),
        out_shape=jax.ShapeDtypeStruct((N, UW), jnp.float32),
    )(wh, e2, e2, psel, esc, eden)


def _tc_dense2(pa0, pa1, wout, a2b, qsel, dsel, e4):
    def body(p0_ref, p1_ref, w_ref, a_ref, q_ref, d_ref, e4_ref,
             wh2_ref, e2b_ref):
        sacc = p0_ref[...] + p1_ref[...]
        num = jnp.dot(sacc, q_ref[...], preferred_element_type=jnp.float32)
        den = jnp.dot(sacc, d_ref[...], preferred_element_type=jnp.float32)
        dinv = 1.0 / jnp.maximum(den[:, :NH], 1e-16)
        scale = jnp.dot(dinv, e4_ref[...], preferred_element_type=jnp.float32)
        hp = num * scale
        hcat = jnp.where(hp > 0, hp, jnp.exp(hp) - 1.0)
        z = jnp.dot(hcat, w_ref[...], preferred_element_type=jnp.float32)
        wh2 = jnp.where(z > 0, z, ALPHA * z)
        wh2_ref[...] = wh2
        e2b_ref[...] = jnp.dot(wh2, a_ref[...], preferred_element_type=jnp.float32)

    return pl.pallas_call(
        body,
        grid=(GRID,),
        in_specs=[
            pl.BlockSpec((RB, UW), lambda i: (i, 0)),
            pl.BlockSpec((RB, UW), lambda i: (i, 0)),
            pl.BlockSpec((F, NCLS), lambda i: (0, 0)),
            pl.BlockSpec((NCLS, 8), lambda i: (0, 0)),
            pl.BlockSpec((UW, F), lambda i: (0, 0)),
            pl.BlockSpec((UW, 8), lambda i: (0, 0)),
            pl.BlockSpec((NH, F), lambda i: (0, 0)),
        ],
        out_specs=[
            pl.BlockSpec((RB, NCLS), lambda i: (i, 0)),
            pl.BlockSpec((RB, 8), lambda i: (i, 0)),
        ],
        out_shape=[
            jax.ShapeDtypeStruct((N, NCLS), jnp.float32),
            jax.ShapeDtypeStruct((N, 8), jnp.float32),
        ],
    )(pa0, pa1, wout, a2b, qsel, dsel, e4)


def _tc_build_v(wh2, e2b, p3, e3, e3den):
    def body(wh2_ref, e2b_ref, e2f_ref, p3_ref, e3_ref, e3d_ref, v_ref):
        cmax = jnp.max(e2f_ref[...][:, 0])
        g = jnp.exp(e2b_ref[...][:, 0:1] - cmax)
        feat = jnp.dot(wh2_ref[...], p3_ref[...],
                       preferred_element_type=jnp.float32)
        scale = jnp.dot(g, e3_ref[...], preferred_element_type=jnp.float32)
        den = jnp.dot(g, e3d_ref[...], preferred_element_type=jnp.float32)
        v_ref[...] = feat * scale + den

    return pl.pallas_call(
        body,
        grid=(GRID,),
        in_specs=[
            pl.BlockSpec((RB, NCLS), lambda i: (i, 0)),
            pl.BlockSpec((RB, 8), lambda i: (i, 0)),
            pl.BlockSpec((N, 8), lambda i: (0, 0)),
            pl.BlockSpec((NCLS, VW), lambda i: (0, 0)),
            pl.BlockSpec((1, VW), lambda i: (0, 0)),
            pl.BlockSpec((1, VW), lambda i: (0, 0)),
        ],
        out_specs=pl.BlockSpec((RB, VW), lambda i: (i, 0)),
        out_shape=jax.ShapeDtypeStruct((N, VW), jnp.float32),
    )(wh2, e2b, e2b, p3, e3, e3den)


def _tc_final(pb0, pb1):
    def body(p0_ref, p1_ref, o_ref):
        sacc = p0_ref[...] + p1_ref[...]
        den = jnp.maximum(sacc[:, NCLS:NCLS + 1], 1e-16)
        o = sacc[:, :NCLS] / den
        o = jnp.where(o > 0, o, jnp.exp(o) - 1.0)
        m = jnp.max(o, axis=1, keepdims=True)
        ex = jnp.exp(o - m)
        o_ref[...] = o - (jnp.log(jnp.sum(ex, axis=1, keepdims=True)) + m)

    return pl.pallas_call(
        body,
        grid=(GRID,),
        in_specs=[
            pl.BlockSpec((RB, VW), lambda i: (i, 0)),
            pl.BlockSpec((RB, VW), lambda i: (i, 0)),
        ],
        out_specs=pl.BlockSpec((RB, NCLS), lambda i: (i, 0)),
        out_shape=jax.ShapeDtypeStruct((N, NCLS), jnp.float32),
    )(pb0, pb1)


def kernel(x, edge_index, W0, a0, W1, a1, W2, a2, W3, a3, W_out, a_out):
    x = x.astype(jnp.float32)
    ei = edge_index.astype(jnp.int32)
    pad = EPAD - E
    sink = N + (jnp.arange(pad, dtype=jnp.int32) % (SROWS - N))
    srcp = jnp.concatenate([ei[0], sink])
    dstp = jnp.concatenate([ei[1], jnp.zeros((pad,), jnp.int32)])
    src_b = srcp.reshape(32, TB, B)
    dst_b = dstp.reshape(32, TB, B)

    wcat = jnp.concatenate([W0, W1, W2, W3], axis=1)
    a2cols = [
        jnp.zeros((F, 1), jnp.float32).at[h * HID:(h + 1) * HID].set(a[HID:2 * HID])
        for h, a in enumerate((a0, a1, a2, a3))
    ]
    a2 = jnp.concatenate(a2cols + [jnp.zeros((F, 4), jnp.float32)], axis=1)
    a2b = jnp.concatenate([a_out[NCLS:2 * NCLS], jnp.zeros((NCLS, 7), jnp.float32)],
                          axis=1)

    psel = np.zeros((F, UW), np.float32)
    psel[:, :F] = np.eye(F)
    esc = np.zeros((8, UW), np.float32)
    eden = np.zeros((8, UW), np.float32)
    for h in range(NH):
        esc[h, h * HID:(h + 1) * HID] = 1.0
        eden[h, F + h] = 1.0
    qsel = np.zeros((UW, F), np.float32)
    qsel[:F, :] = np.eye(F)
    dsel = np.zeros((UW, 8), np.float32)
    for h in range(NH):
        dsel[F + h, h] = 1.0
    e4 = np.zeros((NH, F), np.float32)
    for h in range(NH):
        e4[h, h * HID:(h + 1) * HID] = 1.0
    p3 = np.zeros((NCLS, VW), np.float32)
    p3[:, :NCLS] = np.eye(NCLS)
    e3 = np.zeros((1, VW), np.float32)
    e3[0, :NCLS] = 1.0
    e3den = np.zeros((1, VW), np.float32)
    e3den[0, NCLS] = 1.0

    wh, e2 = _tc_dense1(x, wcat, a2)
    u = _tc_build_u(wh, e2, jnp.asarray(psel), jnp.asarray(esc),
                    jnp.asarray(eden))
    pa0, pa1 = _sc_scatter_a(u, src_b, dst_b)
    wh2, e2b = _tc_dense2(pa0, pa1, W_out, a2b, jnp.asarray(qsel),
                          jnp.asarray(dsel), jnp.asarray(e4))
    v = _tc_build_v(wh2, e2b, jnp.asarray(p3), jnp.asarray(e3),
                    jnp.asarray(e3den))
    pb0, pb1 = _sc_scatter_b(v, src_b, dst_b)
    return _tc_final(pb0, pb1)


# speedup vs baseline: 1.2652x; 1.2652x over previous
import functools

import numpy as np

import jax
import jax.numpy as jnp
from jax import lax
from jax.experimental import pallas as pl
from jax.experimental.pallas import tpu as pltpu
import jax.experimental.pallas.tpu_sc as plsc

N = 10000
F = 128
HID = 32
NH = 4
NCLS = 32
ALPHA = 0.2
E = 320000

UW = 80
VW = 48

B = 128
SROWS = 10112

TA = 160
TB = 80
EPAD = 16 * TA * B

RB = 400
GRID = N // RB

ZROWS = SROWS // 16
OLAST = N - 15 * ZROWS


def _zero_fill(buf, acc, s):
    width = buf.shape[1]

    def zrow(r, carry):
        def zcol(j, carry2):
            buf[r, pl.ds(j * 16, 16)] = jnp.zeros((16,), jnp.float32)
            return carry2
        return lax.fori_loop(0, width // 16, zcol, carry)
    lax.fori_loop(0, B, zrow, 0)

    zbase = s * ZROWS
    for k in range(ZROWS // B):
        pltpu.sync_copy(buf, acc.at[pl.ds(zbase + k * B, B)])
    rem = ZROWS % B
    if rem:
        pltpu.sync_copy(buf.at[pl.ds(0, rem)],
                        acc.at[pl.ds(zbase + (ZROWS // B) * B, rem)])


def _edge_loop(table, acc, src_v, dst_v, bufs, sems, steps):
    nb = len(bufs)
    for b in range(nb):
        pltpu.async_copy(table.at[dst_v.at[b]], bufs[b], sems[b])

    def body(g, carry):
        for b in range(nb):
            t = g * nb + b
            pltpu.make_async_copy(table.at[dst_v.at[t]], bufs[b],
                                  sems[b]).wait()
            pltpu.sync_copy(bufs[b], acc.at[src_v.at[t]], add=True)

            @pl.when(t + nb < steps)
            def _():
                pltpu.async_copy(table.at[dst_v.at[t + nb]], bufs[b], sems[b])
        return carry
    lax.fori_loop(0, steps // nb, body, 0)


def _copy_out(acc, out, s):
    @pl.when(s < 15)
    def _():
        ob = s * ZROWS
        pltpu.sync_copy(acc.at[pl.ds(ob, ZROWS)], out.at[pl.ds(ob, ZROWS)])

    @pl.when(s == 15)
    def _():
        pltpu.sync_copy(acc.at[pl.ds(15 * ZROWS, OLAST)],
                        out.at[pl.ds(15 * ZROWS, OLAST)])


def _sc_pass_a():
    mesh = plsc.VectorSubcoreMesh(core_axis_name="c", subcore_axis_name="s")
    out_sd = jax.ShapeDtypeStruct((N, UW), jnp.float32)

    @functools.partial(
        pl.kernel,
        out_type=(out_sd, out_sd),
        mesh=mesh,
        scratch_types=[
            pltpu.VMEM((TA, B), jnp.int32),
            pltpu.VMEM((TA, B), jnp.int32),
            pltpu.VMEM((B, UW), jnp.float32),
            pltpu.VMEM((B, UW), jnp.float32),
            pltpu.VMEM_SHARED((SROWS, UW), jnp.float32),
            pltpu.SemaphoreType.DMA,
            pltpu.SemaphoreType.DMA,
        ],
        compiler_params=pltpu.CompilerParams(use_tc_tiling_on_sc=False),
    )
    def sc_a(u0, u1, src_hbm, dst_hbm, out0, out1, src_v, dst_v, bufa, bufb,
             acc, sema, semb):
        c = lax.axis_index("c")
        s = lax.axis_index("s")

        _zero_fill(bufa, acc, s)
        plsc.subcore_barrier()

        pltpu.sync_copy(src_hbm.at[s], src_v)
        pltpu.sync_copy(dst_hbm.at[s], dst_v)

        @pl.when(c == 0)
        def _():
            _edge_loop(u0, acc, src_v, dst_v, (bufa, bufb), (sema, semb), TA)

        @pl.when(c == 1)
        def _():
            _edge_loop(u1, acc, src_v, dst_v, (bufa, bufb), (sema, semb), TA)

        plsc.subcore_barrier()

        @pl.when(c == 0)
        def _():
            _copy_out(acc, out0, s)

        @pl.when(c == 1)
        def _():
            _copy_out(acc, out1, s)

    return sc_a


def _sc_pass_b():
    mesh = plsc.VectorSubcoreMesh(core_axis_name="c", subcore_axis_name="s")
    out_sd = jax.ShapeDtypeStruct((N, VW), jnp.float32)

    @functools.partial(
        pl.kernel,
        out_type=(out_sd, out_sd),
        mesh=mesh,
        scratch_types=[
            pltpu.VMEM((TB, B), jnp.int32),
            pltpu.VMEM((TB, B), jnp.int32),
            pltpu.VMEM((B, VW), jnp.float32),
            pltpu.VMEM((B, VW), jnp.float32),
            pltpu.VMEM((B, VW), jnp.float32),
            pltpu.VMEM((B, VW), jnp.float32),
            pltpu.VMEM_SHARED((SROWS, VW), jnp.float32),
            pltpu.SemaphoreType.DMA,
            pltpu.SemaphoreType.DMA,
            pltpu.SemaphoreType.DMA,
            pltpu.SemaphoreType.DMA,
        ],
        compiler_params=pltpu.CompilerParams(use_tc_tiling_on_sc=False),
    )
    def sc_b(table, src_hbm, dst_hbm, out0, out1, src_v, dst_v, b0, b1, b2, b3,
             acc, s0, s1, s2, s3):
        c = lax.axis_index("c")
        s = lax.axis_index("s")
        wid = c * 16 + s

        _zero_fill(b0, acc, s)
        plsc.subcore_barrier()

        pltpu.sync_copy(src_hbm.at[wid], src_v)
        pltpu.sync_copy(dst_hbm.at[wid], dst_v)

        _edge_loop(table, acc, src_v, dst_v, (b0, b1, b2, b3),
                   (s0, s1, s2, s3), TB)

        plsc.subcore_barrier()

        @pl.when(c == 0)
        def _():
            _copy_out(acc, out0, s)

        @pl.when(c == 1)
        def _():
            _copy_out(acc, out1, s)

    return sc_b


_sc_scatter_a = _sc_pass_a()
_sc_scatter_b = _sc_pass_b()


def _tc_dense1(x, wcat, a2):
    def body(x_ref, w_ref, a_ref, wh_ref, e2_ref):
        z = jnp.dot(x_ref[...], w_ref[...], preferred_element_type=jnp.float32)
        wh = jnp.where(z > 0, z, ALPHA * z)
        wh_ref[...] = wh
        e2_ref[...] = jnp.dot(wh, a_ref[...], preferred_element_type=jnp.float32)

    return pl.pallas_call(
        body,
        grid=(GRID,),
        in_specs=[
            pl.BlockSpec((RB, F), lambda i: (i, 0)),
            pl.BlockSpec((F, F), lambda i: (0, 0)),
            pl.BlockSpec((F, 8), lambda i: (0, 0)),
        ],
        out_specs=[
            pl.BlockSpec((RB, F), lambda i: (i, 0)),
            pl.BlockSpec((RB, 8), lambda i: (i, 0)),
        ],
        out_shape=[
            jax.ShapeDtypeStruct((N, F), jnp.float32),
            jax.ShapeDtypeStruct((N, 8), jnp.float32),
        ],
    )(x, wcat, a2)


def _tc_build_u(wh, e2, p0, p1, esc0, esc1, eden0, eden1):
    def body(wh_ref, e2b_ref, e2f_ref, p0_ref, p1_ref, esc0_ref, esc1_ref,
             eden0_ref, eden1_ref, u0_ref, u1_ref):
        cmax = jnp.max(e2f_ref[...], axis=0)
        g = jnp.exp(e2b_ref[...] - cmax[None, :])
        wh_v = wh_ref[...]
        for p_ref, esc_ref, eden_ref, u_ref in (
                (p0_ref, esc0_ref, eden0_ref, u0_ref),
                (p1_ref, esc1_ref, eden1_ref, u1_ref)):
            feat = jnp.dot(wh_v, p_ref[...], preferred_element_type=jnp.float32)
            scale = jnp.dot(g, esc_ref[...], preferred_element_type=jnp.float32)
            den = jnp.dot(g, eden_ref[...], preferred_element_type=jnp.float32)
            u_ref[...] = feat * scale + den

    return pl.pallas_call(
        body,
        grid=(GRID,),
        in_specs=[
            pl.BlockSpec((RB, F), lambda i: (i, 0)),
            pl.BlockSpec((RB, 8), lambda i: (i, 0)),
            pl.BlockSpec((N, 8), lambda i: (0, 0)),
            pl.BlockSpec((F, UW), lambda i: (0, 0)),
            pl.BlockSpec((F, UW), lambda i: (0, 0)),
            pl.BlockSpec((8, UW), lambda i: (0, 0)),
            pl.BlockSpec((8, UW), lambda i: (0, 0)),
            pl.BlockSpec((8, UW), lambda i: (0, 0)),
            pl.BlockSpec((8, UW), lambda i: (0, 0)),
        ],
        out_specs=[
            pl.BlockSpec((RB, UW), lambda i: (i, 0)),
            pl.BlockSpec((RB, UW), lambda i: (i, 0)),
        ],
        out_shape=[
            jax.ShapeDtypeStruct((N, UW), jnp.float32),
            jax.ShapeDtypeStruct((N, UW), jnp.float32),
        ],
    )(wh, e2, e2, p0, p1, esc0, esc1, eden0, eden1)


def _tc_dense2(pa0, pa1, wout, a2b, q0, q1, dsel, e4):
    def body(p0_ref, p1_ref, w_ref, a_ref, q0_ref, q1_ref, d_ref, e4_ref,
             wh2_ref, e2b_ref):
        s0 = p0_ref[...]
        s1 = p1_ref[...]
        num = (jnp.dot(s0, q0_ref[...], preferred_element_type=jnp.float32)
               + jnp.dot(s1, q1_ref[...], preferred_element_type=jnp.float32))
        den = (jnp.dot(s0, d_ref[...][:, :4], preferred_element_type=jnp.float32)
               + jnp.dot(s1, d_ref[...][:, 4:], preferred_element_type=jnp.float32))
        dinv = 1.0 / jnp.maximum(den, 1e-16)
        scale = jnp.dot(dinv, e4_ref[...], preferred_element_type=jnp.float32)
        hp = num * scale
        hcat = jnp.where(hp > 0, hp, jnp.exp(hp) - 1.0)
        z = jnp.dot(hcat, w_ref[...], preferred_element_type=jnp.float32)
        wh2 = jnp.where(z > 0, z, ALPHA * z)
        wh2_ref[...] = wh2
        e2b_ref[...] = jnp.dot(wh2, a_ref[...], preferred_element_type=jnp.float32)

    return pl.pallas_call(
        body,
        grid=(GRID,),
        in_specs=[
            pl.BlockSpec((RB, UW), lambda i: (i, 0)),
            pl.BlockSpec((RB, UW), lambda i: (i, 0)),
            pl.BlockSpec((F, NCLS), lambda i: (0, 0)),
            pl.BlockSpec((NCLS, 8), lambda i: (0, 0)),
            pl.BlockSpec((UW, F), lambda i: (0, 0)),
            pl.BlockSpec((UW, F), lambda i: (0, 0)),
            pl.BlockSpec((UW, 8), lambda i: (0, 0)),
            pl.BlockSpec((NH, F), lambda i: (0, 0)),
        ],
        out_specs=[
            pl.BlockSpec((RB, NCLS), lambda i: (i, 0)),
            pl.BlockSpec((RB, 8), lambda i: (i, 0)),
        ],
        out_shape=[
            jax.ShapeDtypeStruct((N, NCLS), jnp.float32),
            jax.ShapeDtypeStruct((N, 8), jnp.float32),
        ],
    )(pa0, pa1, wout, a2b, q0, q1, dsel, e4)


def _tc_build_v(wh2, e2b, p3, e3, e3den):
    def body(wh2_ref, e2b_ref, e2f_ref, p3_ref, e3_ref, e3d_ref, v_ref):
        cmax = jnp.max(e2f_ref[...][:, 0])
        g = jnp.exp(e2b_ref[...][:, 0:1] - cmax)
        feat = jnp.dot(wh2_ref[...], p3_ref[...],
                       preferred_element_type=jnp.float32)
        scale = jnp.dot(g, e3_ref[...], preferred_element_type=jnp.float32)
        den = jnp.dot(g, e3d_ref[...], preferred_element_type=jnp.float32)
        v_ref[...] = feat * scale + den

    return pl.pallas_call(
        body,
        grid=(GRID,),
        in_specs=[
            pl.BlockSpec((RB, NCLS), lambda i: (i, 0)),
            pl.BlockSpec((RB, 8), lambda i: (i, 0)),
            pl.BlockSpec((N, 8), lambda i: (0, 0)),
            pl.BlockSpec((NCLS, VW), lambda i: (0, 0)),
            pl.BlockSpec((1, VW), lambda i: (0, 0)),
            pl.BlockSpec((1, VW), lambda i: (0, 0)),
        ],
        out_specs=pl.BlockSpec((RB, VW), lambda i: (i, 0)),
        out_shape=jax.ShapeDtypeStruct((N, VW), jnp.float32),
    )(wh2, e2b, e2b, p3, e3, e3den)


def _tc_final(pb0, pb1):
    def body(p0_ref, p1_ref, o_ref):
        sacc = p0_ref[...] + p1_ref[...]
        den = jnp.maximum(sacc[:, NCLS:NCLS + 1], 1e-16)
        o = sacc[:, :NCLS] / den
        o = jnp.where(o > 0, o, jnp.exp(o) - 1.0)
        m = jnp.max(o, axis=1, keepdims=True)
        ex = jnp.exp(o - m)
        o_ref[...] = o - (jnp.log(jnp.sum(ex, axis=1, keepdims=True)) + m)

    return pl.pallas_call(
        body,
        grid=(GRID,),
        in_specs=[
            pl.BlockSpec((RB, VW), lambda i: (i, 0)),
            pl.BlockSpec((RB, VW), lambda i: (i, 0)),
        ],
        out_specs=pl.BlockSpec((RB, NCLS), lambda i: (i, 0)),
        out_shape=jax.ShapeDtypeStruct((N, NCLS), jnp.float32),
    )(pb0, pb1)


def kernel(x, edge_index, W0, a0, W1, a1, W2, a2, W3, a3, W_out, a_out):
    x = x.astype(jnp.float32)
    ei = edge_index.astype(jnp.int32)
    pad = EPAD - E
    sink = N + (jnp.arange(pad, dtype=jnp.int32) % (SROWS - N))
    srcp = jnp.concatenate([ei[0], sink])
    dstp = jnp.concatenate([ei[1], jnp.zeros((pad,), jnp.int32)])
    src_a = srcp.reshape(16, TA, B)
    dst_a = dstp.reshape(16, TA, B)
    src_b = srcp.reshape(32, TB, B)
    dst_b = dstp.reshape(32, TB, B)

    wcat = jnp.concatenate([W0, W1, W2, W3], axis=1)
    a2cols = [
        jnp.zeros((F, 1), jnp.float32).at[h * HID:(h + 1) * HID].set(a[HID:2 * HID])
        for h, a in enumerate((a0, a1, a2, a3))
    ]
    a2 = jnp.concatenate(a2cols + [jnp.zeros((F, 4), jnp.float32)], axis=1)
    a2b = jnp.concatenate([a_out[NCLS:2 * NCLS], jnp.zeros((NCLS, 7), jnp.float32)],
                          axis=1)

    p0 = np.zeros((F, UW), np.float32)
    p0[0:64, 0:64] = np.eye(64)
    p1 = np.zeros((F, UW), np.float32)
    p1[64:128, 0:64] = np.eye(64)
    esc0 = np.zeros((8, UW), np.float32)
    esc0[0, 0:HID] = 1.0
    esc0[1, HID:2 * HID] = 1.0
    esc1 = np.zeros((8, UW), np.float32)
    esc1[2, 0:HID] = 1.0
    esc1[3, HID:2 * HID] = 1.0
    eden0 = np.zeros((8, UW), np.float32)
    eden0[0, 64] = 1.0
    eden0[1, 65] = 1.0
    eden1 = np.zeros((8, UW), np.float32)
    eden1[2, 64] = 1.0
    eden1[3, 65] = 1.0
    q0 = np.zeros((UW, F), np.float32)
    q0[0:64, 0:64] = np.eye(64)
    q1 = np.zeros((UW, F), np.float32)
    q1[0:64, 64:128] = np.eye(64)
    dsel = np.zeros((UW, 8), np.float32)
    dsel[64, 0] = 1.0
    dsel[65, 1] = 1.0
    dsel[64, 6] = 1.0
    dsel[65, 7] = 1.0
    e4 = np.zeros((NH, F), np.float32)
    for h in range(NH):
        e4[h, h * HID:(h + 1) * HID] = 1.0
    p3 = np.zeros((NCLS, VW), np.float32)
    p3[:, :NCLS] = np.eye(NCLS)
    e3 = np.zeros((1, VW), np.float32)
    e3[0, :NCLS] = 1.0
    e3den = np.zeros((1, VW), np.float32)
    e3den[0, NCLS] = 1.0

    wh, e2 = _tc_dense1(x, wcat, a2)
    u0, u1 = _tc_build_u(wh, e2, jnp.asarray(p0), jnp.asarray(p1),
                         jnp.asarray(esc0), jnp.asarray(esc1),
                         jnp.asarray(eden0), jnp.asarray(eden1))
    pa0, pa1 = _sc_scatter_a(u0, u1, src_a, dst_a)
    wh2, e2b = _tc_dense2(pa0, pa1, W_out, a2b, jnp.asarray(q0),
                          jnp.asarray(q1), jnp.asarray(dsel), jnp.asarray(e4))
    v = _tc_build_v(wh2, e2b, jnp.asarray(p3), jnp.asarray(e3),
                    jnp.asarray(e3den))
    pb0, pb1 = _sc_scatter_b(v, src_b, dst_b)
    return _tc_final(pb0, pb1)
